# speedup vs baseline: 3.8380x; 3.8380x over previous
import functools

import jax
import jax.numpy as jnp
from jax import lax
from jax.experimental import pallas as pl
from jax.experimental.pallas import tpu as pltpu
from jax.experimental.pallas import tpu_sc as plsc

D_MODEL = 128
MAX_LEN = 1000

_NUM_CORES = 2
_NUM_SUBCORES = 16
_NUM_WORKERS = _NUM_CORES * _NUM_SUBCORES
_LANES = 16

_CHUNK = 128


def _sc_body(x_hbm, ts_hbm, pe_hbm, out_hbm, idx_all, xbuf, perows, sem,
             *, chunks_per_worker):
    wid = lax.axis_index("s") * _NUM_CORES + lax.axis_index("c")
    idx_row0 = wid * chunks_per_worker
    row0 = idx_row0 * _CHUNK

    pltpu.sync_copy(ts_hbm.at[pl.ds(idx_row0, chunks_per_worker)], idx_all)

    def chunk_body(c, carry):
        for j in range(_CHUNK // _LANES):
            v = idx_all[c, pl.ds(j * _LANES, _LANES)]
            idx_all[c, pl.ds(j * _LANES, _LANES)] = jnp.minimum(
                jnp.maximum(v, 0), MAX_LEN - 1)
        gat = pltpu.async_copy(pe_hbm.at[idx_all.at[c]], perows, sem)
        base = row0 + c * _CHUNK
        pltpu.sync_copy(x_hbm.at[pl.ds(base, _CHUNK)], xbuf)
        gat.wait()

        def add_row(r, carry2):
            for j in range(D_MODEL // _LANES):
                s = pl.ds(j * _LANES, _LANES)
                xbuf[r, s] = xbuf[r, s] + perows[r, s]
            return carry2

        lax.fori_loop(0, _CHUNK, add_row, 0, unroll=False)
        pltpu.sync_copy(xbuf, out_hbm.at[pl.ds(base, _CHUNK)])
        return carry

    lax.fori_loop(0, chunks_per_worker, chunk_body, 0, unroll=False)


def kernel(x, timestamps, pe):
    batch, seq, d = x.shape
    n = batch * seq
    assert d == D_MODEL and n % (_NUM_WORKERS * _CHUNK) == 0
    chunks_per_worker = n // (_NUM_WORKERS * _CHUNK)

    x2 = x.reshape(n, d)
    ts2 = timestamps.astype(jnp.int32).reshape(n // _CHUNK, _CHUNK)

    mesh = plsc.VectorSubcoreMesh(core_axis_name="c", subcore_axis_name="s")
    body = functools.partial(_sc_body, chunks_per_worker=chunks_per_worker)
    out = pl.kernel(
        body,
        out_type=jax.ShapeDtypeStruct((n, d), jnp.float32),
        mesh=mesh,
        scratch_types=[
            pltpu.VMEM((chunks_per_worker, _CHUNK), jnp.int32),
            pltpu.VMEM((_CHUNK, D_MODEL), jnp.float32),
            pltpu.VMEM((_CHUNK, D_MODEL), jnp.float32),
            pltpu.SemaphoreType.DMA,
        ],
    )(x2, ts2, pe)
    return out.reshape(batch, seq, d)
